# baseline (speedup 1.0000x reference)
import jax
import jax.numpy as jnp
from jax import lax
from jax.experimental import pallas as pl
from jax.experimental.pallas import tpu as pltpu
from jax.experimental.pallas import tpu_sc as plsc

_NUM_CORES = 2
_NUM_SUBCORES = 16
_NW = _NUM_CORES * _NUM_SUBCORES
_CHUNK = 128
_RING = 3


def _emb_body(table_hbm, idx_hbm, out_hbm, tshared, *rest):
    rbufs = rest[:_RING]
    ibufs = rest[_RING:2 * _RING]
    gsems = rest[2 * _RING:3 * _RING]
    ssems = rest[3 * _RING:4 * _RING]
    isems = rest[4 * _RING:5 * _RING]

    sid = lax.axis_index("s")
    wid = sid * _NUM_CORES + lax.axis_index("c")
    chunks_per_w = idx_hbm.shape[0] // _NW
    npair = chunks_per_w // 2
    chunk_base = wid * chunks_per_w

    @pl.when(sid == 0)
    def _():
        pltpu.sync_copy(table_hbm, tshared)
    plsc.subcore_barrier()

    def idxc(q, s):
        return pltpu.make_async_copy(
            idx_hbm.at[pl.ds(chunk_base + 2 * q, 2)], ibufs[s], isems[s])

    def gath(q, s, h):
        del q
        return pltpu.make_async_copy(
            tshared.at[ibufs[s].at[h]],
            rbufs[s].at[pl.ds(h * _CHUNK, _CHUNK)], gsems[s])

    def store(q, s):
        off = (chunk_base + 2 * q) * _CHUNK
        return pltpu.make_async_copy(
            rbufs[s], out_hbm.at[pl.ds(off, 2 * _CHUNK)], ssems[s])

    def step(q, s, wait_store=True, fire_ahead=True, guard_idx=False):
        sp = (s + 2) % _RING
        gath(q, s, 0).wait()
        gath(q, s, 1).wait()
        store(q, s).start()
        if wait_store:
            store(q - 1, sp).wait()
        if fire_ahead:
            idxc(q + 2, sp).wait()
            gath(q + 2, sp, 0).start()
            gath(q + 2, sp, 1).start()
            if guard_idx:
                @pl.when(q + 3 < npair)
                def _():
                    idxc(q + 3, s).start()
            else:
                idxc(q + 3, s).start()

    idxc(0, 0).start()
    idxc(1, 1).start()
    idxc(2, 2).start()
    idxc(0, 0).wait()
    gath(0, 0, 0).start()
    gath(0, 0, 1).start()
    idxc(1, 1).wait()
    gath(1, 1, 0).start()
    gath(1, 1, 1).start()
    step(0, 0, wait_store=False)
    step(1, 1)

    def body(m, carry):
        q = 3 * m + 2
        step(q, 2, guard_idx=True)
        step(q + 1, 0, guard_idx=True)
        step(q + 2, 1, guard_idx=True)
        return carry

    lax.fori_loop(0, (npair - 4) // 3, body, 0)

    step(npair - 2, (npair - 2) % _RING, fire_ahead=False)
    step(npair - 1, (npair - 1) % _RING, fire_ahead=False)
    store(npair - 1, (npair - 1) % _RING).wait()


def kernel(x, table):
    b, h = x.shape
    v, d = table.shape
    n = b * h
    idx = x.reshape(n // _CHUNK, _CHUNK).astype(jnp.int32)

    mesh = plsc.VectorSubcoreMesh(
        core_axis_name="c",
        subcore_axis_name="s",
        num_cores=_NUM_CORES,
        num_subcores=_NUM_SUBCORES,
    )
    k = pl.kernel(
        _emb_body,
        out_type=jax.ShapeDtypeStruct((n, d), table.dtype),
        mesh=mesh,
        scratch_types=(
            [pltpu.VMEM_SHARED((v, d), jnp.float32)]
            + [pltpu.VMEM((2 * _CHUNK, d), jnp.float32)] * _RING
            + [pltpu.VMEM((2, _CHUNK), jnp.int32)] * _RING
            + [pltpu.SemaphoreType.DMA] * (3 * _RING)
        ),
    )
    out = k(table, idx)
    return out.reshape(b, h, d)
